# Initial kernel scaffold; baseline (speedup 1.0000x reference)
import jax
import jax.numpy as jnp
from jax.experimental import pallas as pl


def kernel(features):
    raise NotImplementedError("write your pallas kernel here")


# speedup vs baseline: 34.3294x; 34.3294x over previous
import functools

import jax
import jax.numpy as jnp
from jax.experimental import pallas as pl

_K = 64
_D = 8192
_ROWS_PER_BLOCK = 128


def _topk_mask_block(x_ref, o_ref):
    x = x_ref[...]
    bits = jax.lax.bitcast_convert_type(x, jnp.int32)
    sgn = jax.lax.shift_right_arithmetic(bits, 31)
    skey = bits ^ (sgn & jnp.int32(0x7FFFFFFF))

    r = x.shape[0]
    def body(i, ku):
        b = 31 - i
        cand_u = ku | (jnp.int32(1) << b)
        scand = cand_u ^ jnp.int32(-0x80000000)
        cnt = jnp.sum((skey >= scand).astype(jnp.float32), axis=1, keepdims=True)
        return jnp.where(cnt >= _K, cand_u, ku)

    ku0 = jnp.zeros((r, 1), jnp.int32)
    ku = jax.lax.fori_loop(0, 32, body, ku0, unroll=True)
    sthr = ku ^ jnp.int32(-0x80000000)
    o_ref[...] = jnp.where(skey >= sthr, x, jnp.float32(0.0))


@jax.jit
def kernel(features):
    B, L, D = features.shape
    x = features.reshape(B * L, D)
    n_rows = B * L
    r = _ROWS_PER_BLOCK
    out = pl.pallas_call(
        _topk_mask_block,
        grid=(n_rows // r,),
        in_specs=[pl.BlockSpec((r, D), lambda i: (i, 0))],
        out_specs=pl.BlockSpec((r, D), lambda i: (i, 0)),
        out_shape=jax.ShapeDtypeStruct((n_rows, D), jnp.float32),
    )(x)
    return out.reshape(B, L, D)
